# baseline (speedup 1.0000x reference)
import functools

import jax
import jax.numpy as jnp
import numpy as np
from jax import lax
from jax.experimental import pallas as pl
from jax.experimental.pallas import tpu as pltpu
from jax.experimental.pallas import tpu_sc as plsc

_NU, _NB, _NI, _D = 20000, 10000, 50000, 64
_NC, _NS = 2, 16
_KB = 128
_BR = 256
_F32 = jnp.float32
_SDS = jax.ShapeDtypeStruct


def _rup(x, m):
    return (x + m - 1) // m * m


def _ra(n):
    return _rup(n + 16, 128)


def _mesh():
    return plsc.VectorSubcoreMesh(core_axis_name="c", subcore_axis_name="s",
                                  num_cores=_NC, num_subcores=_NS)


_SC_PARAMS = pltpu.CompilerParams(use_tc_tiling_on_sc=False,
                                  needs_layout_passes=False)


@functools.lru_cache(maxsize=None)
def _hist_pair(n_a, n_b, e_pad):
    pair = n_b is not None
    ra = _ra(n_a)
    za = ra // 16
    if pair:
        rb = _ra(n_b)
        zb = rb // 16
        out_type = (_SDS((2 * ra, 16), _F32), _SDS((2 * rb, 16), _F32))
    else:
        out_type = _SDS((2 * ra, 16), _F32)
    npt = e_pad // _KB // (_NC * _NS)
    assert npt % 2 == 0 and npt >= 2

    scratch = [
        pltpu.VMEM((_KB,), jnp.int32),
        pltpu.VMEM((_KB,), jnp.int32),
        pltpu.VMEM((_KB, 16), _F32),
        pltpu.VMEM_SHARED((ra, 16), _F32),
        pltpu.SemaphoreType.DMA,
        pltpu.SemaphoreType.DMA,
    ]
    if pair:
        scratch += [
            pltpu.VMEM((_KB,), jnp.int32),
            pltpu.VMEM((_KB,), jnp.int32),
            pltpu.VMEM_SHARED((rb, 16), _F32),
        ]

    @functools.partial(
        pl.kernel,
        out_type=out_type,
        mesh=_mesh(),
        compiler_params=_SC_PARAMS,
        scratch_types=scratch,
    )
    def k(*refs):
        ib_hbm = ob_hbm = ib0 = ib1 = acc_b = None
        if pair:
            (ia_hbm, ib_hbm, ones_hbm, z16_hbm, oa_hbm, ob_hbm,
             ia0, ia1, ones_v, acc_a, sem0, sem1, ib0, ib1, acc_b) = refs
        else:
            (ia_hbm, ones_hbm, z16_hbm, oa_hbm,
             ia0, ia1, ones_v, acc_a, sem0, sem1) = refs
        c = lax.axis_index("c")
        s = lax.axis_index("s")
        pltpu.sync_copy(ones_hbm, ones_v)
        pltpu.sync_copy(z16_hbm.at[pl.ds(0, za)], acc_a.at[pl.ds(s * za, za)])
        if pair:
            pltpu.sync_copy(z16_hbm.at[pl.ds(0, zb)],
                            acc_b.at[pl.ds(s * zb, zb)])
        plsc.subcore_barrier()
        base = (c * _NS + s) * npt

        def load(blk, ia, ib):
            pltpu.sync_copy(ia_hbm.at[pl.ds(blk * _KB, _KB)], ia)
            if pair:
                pltpu.sync_copy(ib_hbm.at[pl.ds(blk * _KB, _KB)], ib)

        def fire(ia, ib, sem):
            pltpu.async_copy(ones_v, acc_a.at[ia], sem, add=True)
            if pair:
                pltpu.async_copy(ones_v, acc_b.at[ib], sem, add=True)

        def drain(ia, ib, sem):
            pltpu.make_async_copy(ones_v, acc_a.at[ia], sem).wait()
            if pair:
                pltpu.make_async_copy(ones_v, acc_b.at[ib], sem).wait()

        load(base, ia0, ib0)
        fire(ia0, ib0, sem0)
        load(base + 1, ia1, ib1)
        fire(ia1, ib1, sem1)

        def body(i, carry):
            b0 = base + 2 * i
            drain(ia0, ib0, sem0)
            load(b0 + 2, ia0, ib0)
            fire(ia0, ib0, sem0)
            drain(ia1, ib1, sem1)
            load(b0 + 3, ia1, ib1)
            fire(ia1, ib1, sem1)
            return carry

        lax.fori_loop(0, npt // 2 - 1, body, 0)
        drain(ia0, ib0, sem0)
        drain(ia1, ib1, sem1)
        plsc.subcore_barrier()
        pltpu.sync_copy(acc_a.at[pl.ds(s * za, za)],
                        oa_hbm.at[pl.ds(c * ra + s * za, za)])
        if pair:
            pltpu.sync_copy(acc_b.at[pl.ds(s * zb, zb)],
                            ob_hbm.at[pl.ds(c * rb + s * zb, zb)])

    return k


@functools.lru_cache(maxsize=None)
def _edge_prop(n_dst, e_pad, chunk):
    if chunk:
        ch = _rup(n_dst // 2 + 16, 128)
        acc_rows = ch + 128
        out_rows = 2 * ch
        npt = e_pad // _KB // _NS
        wr = ch // 16
    else:
        acc_rows = _ra(n_dst)
        out_rows = 2 * acc_rows
        npt = e_pad // _KB // (_NC * _NS)
        wr = acc_rows // 16
    zr = acc_rows // 16

    nbuf = 2
    assert npt % nbuf == 0 and npt >= nbuf

    @functools.partial(
        pl.kernel,
        out_type=_SDS((out_rows, _D), _F32),
        mesh=_mesh(),
        compiler_params=_SC_PARAMS,
        scratch_types=(
            [pltpu.VMEM((_KB,), jnp.int32)] * nbuf
            + [pltpu.VMEM((_KB,), jnp.int32)] * nbuf
            + [pltpu.VMEM((_KB, _D), jnp.bfloat16)] * nbuf
            + [pltpu.VMEM((_KB, _D), _F32)] * nbuf
            + [pltpu.VMEM_SHARED((acc_rows, _D), _F32)]
            + [pltpu.SemaphoreType.DMA] * nbuf
        ),
    )
    def k(g_hbm, src_hbm, dst_hbm, z_hbm, out_hbm, *scr):
        srcs = scr[0:nbuf]
        dsts = scr[nbuf:2 * nbuf]
        rbfs = scr[2 * nbuf:3 * nbuf]
        rows = scr[3 * nbuf:4 * nbuf]
        acc = scr[4 * nbuf]
        sems = scr[4 * nbuf + 1:]
        c = lax.axis_index("c")
        s = lax.axis_index("s")
        pltpu.sync_copy(z_hbm.at[pl.ds(0, zr)], acc.at[pl.ds(s * zr, zr)])
        plsc.subcore_barrier()
        if chunk:
            base = s * npt
            lo = c * ch
        else:
            base = (c * _NS + s) * npt
        last = base + npt - 1

        def load_start(blk, j):
            pltpu.sync_copy(src_hbm.at[pl.ds(blk * _KB, _KB)], srcs[j])
            pltpu.sync_copy(dst_hbm.at[pl.ds(blk * _KB, _KB)], dsts[j])
            pltpu.async_copy(g_hbm.at[srcs[j]], rbfs[j], sems[j])

        def gwait(j):
            pltpu.make_async_copy(g_hbm.at[pl.ds(0, _KB)], rbfs[j],
                                  sems[j]).wait()

        def scat(j):
            dv = dsts[j]
            rbf = rbfs[j]
            rf = rows[j]

            def crow(i, carry):
                r0 = i * 8
                for rr in range(8):
                    for cc in range(_D // 32):
                        x = rbf[r0 + rr, pl.ds(cc * 32, 32)]
                        a, b = plsc.unpack(
                            x, format=plsc.PackFormat.INTERLEAVED)
                        rf[r0 + rr, pl.ds(cc * 32, 16)] = a
                        rf[r0 + rr, pl.ds(cc * 32 + 16, 16)] = b
                return carry

            lax.fori_loop(0, _KB // 8, crow, 0)
            if chunk:
                for q in range(_KB // 16):
                    dq = dv[pl.ds(q * 16, 16)]
                    lq = dq - lo
                    ok = (lq >= 0) & (lq < ch)
                    dv[pl.ds(q * 16, 16)] = jnp.where(ok, lq, ch + (dq & 15))
            pltpu.sync_copy(rf, acc.at[dv], add=True)

        for j in range(nbuf):
            load_start(base + j, j)

        def body(i, carry):
            b0 = base + nbuf * i
            for j in range(nbuf):
                gwait(j)
                scat(j)
                load_start(jnp.minimum(b0 + nbuf + j, last), j)
            return carry

        lax.fori_loop(0, npt // nbuf, body, 0)
        for j in range(nbuf):
            gwait(j)
        plsc.subcore_barrier()
        obase = c * (ch if chunk else acc_rows)
        pltpu.sync_copy(acc.at[pl.ds(s * wr, wr)],
                        out_hbm.at[pl.ds(obase + s * wr, wr)])

    return k


def _deg2_spec():
    return pl.BlockSpec((2, _BR, 16), lambda i: (0, i, 0))


def _s_spec(p):
    return pl.BlockSpec((p, _BR, _D), lambda i: (0, i, 0))


def _row_spec():
    return pl.BlockSpec((_BR, _D), lambda i: (i, 0))


def _r_of(deg2_ref):
    d = deg2_ref[0, :, 0:1] + deg2_ref[1, :, 0:1]
    return 1.0 / (jnp.sqrt(d) + 1e-8)


def _scale0(deg2, feat, out_rows):
    def body(deg2_ref, f_ref, o_ref):
        o_ref[...] = f_ref[...] * _r_of(deg2_ref)

    return pl.pallas_call(
        body,
        grid=(out_rows // _BR,),
        in_specs=[_deg2_spec(), _row_spec()],
        out_specs=_row_spec(),
        out_shape=_SDS((out_rows, _D), _F32),
    )(deg2, feat)


def _scale0_diff(deg2, fa, fb, out_rows):
    def body(deg2_ref, fa_ref, fb_ref, o_ref):
        o_ref[...] = (fa_ref[...] - fb_ref[...]) * _r_of(deg2_ref)

    return pl.pallas_call(
        body,
        grid=(out_rows // _BR,),
        in_specs=[_deg2_spec(), _row_spec(), _row_spec()],
        out_specs=_row_spec(),
        out_shape=_SDS((out_rows, _D), _F32),
    )(deg2, fa, fb)


def _sum_parts(s_ref):
    p = s_ref.shape[0]
    acc = s_ref[0]
    for j in range(1, p):
        acc = acc + s_ref[j]
    return acc


def _update1(deg2, s, f0_args, out_rows, diff):
    p = s.shape[0]

    def body(*refs):
        if diff:
            deg2_ref, s_ref, fa_ref, fb_ref, t_ref, g_ref = refs
            f0 = fa_ref[...] - fb_ref[...]
        else:
            deg2_ref, s_ref, fa_ref, t_ref, g_ref = refs
            f0 = fa_ref[...]
        r = _r_of(deg2_ref)
        f1 = r * _sum_parts(s_ref) * 0.5
        nrm = jnp.sqrt(jnp.sum(f1 * f1, axis=1, keepdims=True))
        t_ref[...] = f0 + f1 / jnp.maximum(nrm, 1e-12)
        g_ref[...] = r * f1

    in_specs = [_deg2_spec(), _s_spec(p)] + [_row_spec()] * len(f0_args)
    return pl.pallas_call(
        body,
        grid=(out_rows // _BR,),
        in_specs=in_specs,
        out_specs=(_row_spec(), _row_spec()),
        out_shape=(_SDS((out_rows, _D), _F32), _SDS((out_rows, _D), _F32)),
    )(deg2, s, *f0_args)


def _update2(deg2, s, t_in, out_rows):
    p = s.shape[0]

    def body(deg2_ref, s_ref, t_ref, o_ref):
        r = _r_of(deg2_ref)
        f2 = r * _sum_parts(s_ref) * (1.0 / 3.0)
        nrm = jnp.sqrt(jnp.sum(f2 * f2, axis=1, keepdims=True))
        o_ref[...] = t_ref[...] + f2 / jnp.maximum(nrm, 1e-12)

    return pl.pallas_call(
        body,
        grid=(out_rows // _BR,),
        in_specs=[_deg2_spec(), _s_spec(p), _row_spec()],
        out_specs=_row_spec(),
        out_shape=_SDS((out_rows, _D), _F32),
    )(deg2, s, t_in)


def _bi_final(deg2, s, out_rows):
    def body(deg2_ref, s_ref, o_ref):
        size = deg2_ref[0, :, 0:1] + deg2_ref[1, :, 0:1] + 1e-8
        o_ref[...] = _sum_parts(s_ref) / size

    return pl.pallas_call(
        body,
        grid=(out_rows // _BR,),
        in_specs=[_deg2_spec(), _s_spec(s.shape[0])],
        out_specs=_row_spec(),
        out_shape=_SDS((out_rows, _D), _F32),
    )(deg2, s)


def _gtable(g):
    n = g.shape[0]
    v = g.reshape(n, 2, 2, 16).transpose(0, 1, 3, 2).reshape(n, _D)
    return v.astype(jnp.bfloat16)


def _pad_dst(idx, n_dst, e_pad):
    p = e_pad - idx.shape[0]
    fill = jnp.asarray(n_dst + np.arange(p) % 16, jnp.int32)
    return jnp.concatenate([idx.astype(jnp.int32), fill])


def _pad_src(idx, n_src, e_pad):
    p = e_pad - idx.shape[0]
    fill = jnp.asarray(np.arange(p) % n_src, jnp.int32)
    return jnp.concatenate([idx.astype(jnp.int32), fill])


def _branch(a_idx, b_idx, n_a, n_b, feat_a, feat_b_args, zeros, z16, ones16,
            chunk_b):
    e = a_idx.shape[0]
    e_pad = _rup(e, 2 * _KB * _NC * _NS)
    ra, rb = _ra(n_a), _ra(n_b)

    a_dst = _pad_dst(a_idx, n_a, e_pad)
    b_dst = _pad_dst(b_idx, n_b, e_pad)
    a_src = _pad_src(a_idx, n_a, e_pad)
    b_src = _pad_src(b_idx, n_b, e_pad)

    ha, hb = _hist_pair(n_a, n_b, e_pad)(a_dst, b_dst, ones16, z16)
    deg2a = ha.reshape(2, ra, 16)
    deg2b = hb.reshape(2, rb, 16)

    rows_a = _rup(ra, _BR)
    if chunk_b:
        ch = _rup(n_b // 2 + 16, 128)
        rows_b = _rup(2 * ch, _BR)
        pb = 1
    else:
        rows_b = _rup(rb, _BR)
        pb = 2

    ek_a = _edge_prop(n_a, e_pad, False)
    ek_b = _edge_prop(n_b, e_pad, chunk_b)

    if len(feat_b_args) == 2:
        g_b = _scale0_diff(deg2b, feat_b_args[0], feat_b_args[1],
                           _rup(n_b, _BR))
    else:
        g_b = _scale0(deg2b, feat_b_args[0], _rup(n_b, _BR))
    g_a = _scale0(deg2a, feat_a, _rup(n_a, _BR))

    s_a = ek_a(_gtable(g_b), b_src, a_dst, zeros).reshape(2, ra, _D)
    s_b = ek_b(_gtable(g_a), a_src, b_dst, zeros)
    s_b = s_b.reshape(pb, s_b.shape[0] // pb, _D)
    t_a, g_a1 = _update1(deg2a, s_a, (feat_a,), rows_a, False)
    t_b, g_b1 = _update1(deg2b, s_b, feat_b_args, rows_b,
                         len(feat_b_args) == 2)

    s_a2 = ek_a(_gtable(g_b1), b_src, a_dst, zeros).reshape(2, ra, _D)
    s_b2 = ek_b(_gtable(g_a1), a_src, b_dst, zeros)
    s_b2 = s_b2.reshape(pb, s_b2.shape[0] // pb, _D)
    tot_a = _update2(deg2a, s_a2, t_a, rows_a)
    tot_b = _update2(deg2b, s_b2, t_b, rows_b)
    return tot_a, tot_b


def kernel(users_feat, bundles_feat, items_feat, items_pop,
           ui_u, ui_i, ub_u, ub_b, bi_b, bi_i):
    zeros = jnp.zeros((3200, _D), _F32)
    z16 = zeros.reshape(-1, 16)
    ones16 = jnp.zeros((_KB, 16), _F32).at[:, 0].set(1.0)

    hist_u, hist_b = _branch(ub_u, ub_b, _NU, _NB, users_feat,
                             (bundles_feat,), zeros, z16, ones16, False)
    aff_u, ui_items = _branch(ui_u, ui_i, _NU, _NI, users_feat,
                              (items_feat, items_pop), zeros, z16, ones16,
                              True)

    e_pad = _rup(bi_b.shape[0], 2 * _KB * _NC * _NS)
    b_dst = _pad_dst(bi_b, _NB, e_pad)
    i_src = _pad_src(bi_i, _NI, e_pad)
    hbi = _hist_pair(_NB, None, e_pad)(b_dst, ones16, z16)
    deg2bi = hbi.reshape(2, _ra(_NB), 16)
    s_bi = _edge_prop(_NB, e_pad, False)(_gtable(ui_items), i_src, b_dst,
                                         zeros)
    s_bi = s_bi.reshape(2, _ra(_NB), _D)
    aff_b = _bi_final(deg2bi, s_bi, _rup(_ra(_NB), _BR))

    return (aff_u[:_NU], hist_u[:_NU], aff_b[:_NB], hist_b[:_NB])


# speedup vs baseline: 1.7808x; 1.7808x over previous
import functools

import jax
import jax.numpy as jnp
import numpy as np
from jax import lax
from jax.experimental import pallas as pl
from jax.experimental.pallas import tpu as pltpu
from jax.experimental.pallas import tpu_sc as plsc

_NU, _NB, _NI, _D = 20000, 10000, 50000, 64
_NC, _NS = 2, 16
_KB = 128
_BR = 256
_F32 = jnp.float32
_SDS = jax.ShapeDtypeStruct


def _rup(x, m):
    return (x + m - 1) // m * m


def _ra(n):
    return _rup(n + 16, 128)


def _mesh():
    return plsc.VectorSubcoreMesh(core_axis_name="c", subcore_axis_name="s",
                                  num_cores=_NC, num_subcores=_NS)


_SC_PARAMS = pltpu.CompilerParams(use_tc_tiling_on_sc=False)


@functools.lru_cache(maxsize=None)
def _hist_pair(n_a, n_b, e_pad):
    pair = n_b is not None
    ra = _ra(n_a)
    za = ra // 16
    if pair:
        rb = _ra(n_b)
        zb = rb // 16
        out_type = (_SDS((2 * ra, 16), _F32), _SDS((2 * rb, 16), _F32))
    else:
        out_type = _SDS((2 * ra, 16), _F32)
    npt = e_pad // _KB // (_NC * _NS)
    assert npt % 2 == 0 and npt >= 2

    scratch = [
        pltpu.VMEM((_KB,), jnp.int32),
        pltpu.VMEM((_KB,), jnp.int32),
        pltpu.VMEM((_KB, 16), _F32),
        pltpu.VMEM_SHARED((ra, 16), _F32),
        pltpu.SemaphoreType.DMA,
        pltpu.SemaphoreType.DMA,
    ]
    if pair:
        scratch += [
            pltpu.VMEM((_KB,), jnp.int32),
            pltpu.VMEM((_KB,), jnp.int32),
            pltpu.VMEM_SHARED((rb, 16), _F32),
        ]

    @functools.partial(
        pl.kernel,
        out_type=out_type,
        mesh=_mesh(),
        compiler_params=_SC_PARAMS,
        scratch_types=scratch,
    )
    def k(*refs):
        ib_hbm = ob_hbm = ib0 = ib1 = acc_b = None
        if pair:
            (ia_hbm, ib_hbm, ones_hbm, z16_hbm, oa_hbm, ob_hbm,
             ia0, ia1, ones_v, acc_a, sem0, sem1, ib0, ib1, acc_b) = refs
        else:
            (ia_hbm, ones_hbm, z16_hbm, oa_hbm,
             ia0, ia1, ones_v, acc_a, sem0, sem1) = refs
        c = lax.axis_index("c")
        s = lax.axis_index("s")
        pltpu.sync_copy(ones_hbm, ones_v)
        pltpu.sync_copy(z16_hbm.at[pl.ds(0, za)], acc_a.at[pl.ds(s * za, za)])
        if pair:
            pltpu.sync_copy(z16_hbm.at[pl.ds(0, zb)],
                            acc_b.at[pl.ds(s * zb, zb)])
        plsc.subcore_barrier()
        base = (c * _NS + s) * npt

        def load(blk, ia, ib):
            pltpu.sync_copy(ia_hbm.at[pl.ds(blk * _KB, _KB)], ia)
            if pair:
                pltpu.sync_copy(ib_hbm.at[pl.ds(blk * _KB, _KB)], ib)

        def fire(ia, ib, sem):
            pltpu.async_copy(ones_v, acc_a.at[ia], sem, add=True)
            if pair:
                pltpu.async_copy(ones_v, acc_b.at[ib], sem, add=True)

        def drain(ia, ib, sem):
            pltpu.make_async_copy(ones_v, acc_a.at[ia], sem).wait()
            if pair:
                pltpu.make_async_copy(ones_v, acc_b.at[ib], sem).wait()

        load(base, ia0, ib0)
        fire(ia0, ib0, sem0)
        load(base + 1, ia1, ib1)
        fire(ia1, ib1, sem1)

        def body(i, carry):
            b0 = base + 2 * i
            drain(ia0, ib0, sem0)
            load(b0 + 2, ia0, ib0)
            fire(ia0, ib0, sem0)
            drain(ia1, ib1, sem1)
            load(b0 + 3, ia1, ib1)
            fire(ia1, ib1, sem1)
            return carry

        lax.fori_loop(0, npt // 2 - 1, body, 0)
        drain(ia0, ib0, sem0)
        drain(ia1, ib1, sem1)
        plsc.subcore_barrier()
        pltpu.sync_copy(acc_a.at[pl.ds(s * za, za)],
                        oa_hbm.at[pl.ds(c * ra + s * za, za)])
        if pair:
            pltpu.sync_copy(acc_b.at[pl.ds(s * zb, zb)],
                            ob_hbm.at[pl.ds(c * rb + s * zb, zb)])

    return k


@functools.lru_cache(maxsize=None)
def _edge_prop(n_dst, e_pad, chunk):
    if chunk:
        ch = _rup(n_dst // 2 + 16, 128)
        acc_rows = ch + 128
        out_rows = 2 * ch
        npt = e_pad // _KB // _NS
        wr = ch // 16
    else:
        acc_rows = _ra(n_dst)
        out_rows = 2 * acc_rows
        npt = e_pad // _KB // (_NC * _NS)
        wr = acc_rows // 16
    zr = acc_rows // 16

    nbuf = 2
    assert npt % nbuf == 0 and npt >= nbuf

    @functools.partial(
        pl.kernel,
        out_type=_SDS((out_rows, _D), _F32),
        mesh=_mesh(),
        compiler_params=_SC_PARAMS,
        scratch_types=(
            [pltpu.VMEM((_KB,), jnp.int32)] * nbuf
            + [pltpu.VMEM((_KB,), jnp.int32)] * nbuf
            + [pltpu.VMEM((_KB, _D), _F32)] * nbuf
            + [pltpu.VMEM_SHARED((acc_rows, _D), _F32)]
            + [pltpu.SemaphoreType.DMA] * nbuf
        ),
    )
    def k(g_hbm, src_hbm, dst_hbm, z_hbm, out_hbm, *scr):
        srcs = scr[0:nbuf]
        dsts = scr[nbuf:2 * nbuf]
        rows = scr[2 * nbuf:3 * nbuf]
        acc = scr[3 * nbuf]
        sems = scr[3 * nbuf + 1:]
        c = lax.axis_index("c")
        s = lax.axis_index("s")
        pltpu.sync_copy(z_hbm.at[pl.ds(0, zr)], acc.at[pl.ds(s * zr, zr)])
        plsc.subcore_barrier()
        if chunk:
            base = s * npt
            lo = c * ch
        else:
            base = (c * _NS + s) * npt
        last = base + npt - 1

        def load_start(blk, j):
            pltpu.sync_copy(src_hbm.at[pl.ds(blk * _KB, _KB)], srcs[j])
            pltpu.sync_copy(dst_hbm.at[pl.ds(blk * _KB, _KB)], dsts[j])
            pltpu.async_copy(g_hbm.at[srcs[j]], rows[j], sems[j])

        def gwait(j):
            pltpu.make_async_copy(g_hbm.at[pl.ds(0, _KB)], rows[j],
                                  sems[j]).wait()

        def scat(j):
            dv = dsts[j]
            rf = rows[j]
            if chunk:
                for q in range(_KB // 16):
                    dq = dv[pl.ds(q * 16, 16)]
                    lq = dq - lo
                    ok = (lq >= 0) & (lq < ch)
                    dv[pl.ds(q * 16, 16)] = jnp.where(ok, lq, ch + (dq & 15))
            pltpu.sync_copy(rf, acc.at[dv], add=True)

        for j in range(nbuf):
            load_start(base + j, j)

        def body(i, carry):
            b0 = base + nbuf * i
            for j in range(nbuf):
                gwait(j)
                scat(j)
                load_start(jnp.minimum(b0 + nbuf + j, last), j)
            return carry

        lax.fori_loop(0, npt // nbuf, body, 0)
        for j in range(nbuf):
            gwait(j)
        plsc.subcore_barrier()
        obase = c * (ch if chunk else acc_rows)
        pltpu.sync_copy(acc.at[pl.ds(s * wr, wr)],
                        out_hbm.at[pl.ds(obase + s * wr, wr)])

    return k


def _deg2_spec():
    return pl.BlockSpec((2, _BR, 16), lambda i: (0, i, 0))


def _s_spec(p):
    return pl.BlockSpec((p, _BR, _D), lambda i: (0, i, 0))


def _row_spec():
    return pl.BlockSpec((_BR, _D), lambda i: (i, 0))


def _r_of(deg2_ref):
    d = deg2_ref[0, :, 0:1] + deg2_ref[1, :, 0:1]
    return 1.0 / (jnp.sqrt(d) + 1e-8)


def _scale0(deg2, feat, out_rows):
    def body(deg2_ref, f_ref, o_ref):
        o_ref[...] = f_ref[...] * _r_of(deg2_ref)

    return pl.pallas_call(
        body,
        grid=(out_rows // _BR,),
        in_specs=[_deg2_spec(), _row_spec()],
        out_specs=_row_spec(),
        out_shape=_SDS((out_rows, _D), _F32),
    )(deg2, feat)


def _scale0_diff(deg2, fa, fb, out_rows):
    def body(deg2_ref, fa_ref, fb_ref, o_ref):
        o_ref[...] = (fa_ref[...] - fb_ref[...]) * _r_of(deg2_ref)

    return pl.pallas_call(
        body,
        grid=(out_rows // _BR,),
        in_specs=[_deg2_spec(), _row_spec(), _row_spec()],
        out_specs=_row_spec(),
        out_shape=_SDS((out_rows, _D), _F32),
    )(deg2, fa, fb)


def _sum_parts(s_ref):
    p = s_ref.shape[0]
    acc = s_ref[0]
    for j in range(1, p):
        acc = acc + s_ref[j]
    return acc


def _update1(deg2, s, f0_args, out_rows, diff):
    p = s.shape[0]

    def body(*refs):
        if diff:
            deg2_ref, s_ref, fa_ref, fb_ref, t_ref, g_ref = refs
            f0 = fa_ref[...] - fb_ref[...]
        else:
            deg2_ref, s_ref, fa_ref, t_ref, g_ref = refs
            f0 = fa_ref[...]
        r = _r_of(deg2_ref)
        f1 = r * _sum_parts(s_ref) * 0.5
        nrm = jnp.sqrt(jnp.sum(f1 * f1, axis=1, keepdims=True))
        t_ref[...] = f0 + f1 / jnp.maximum(nrm, 1e-12)
        g_ref[...] = r * f1

    in_specs = [_deg2_spec(), _s_spec(p)] + [_row_spec()] * len(f0_args)
    return pl.pallas_call(
        body,
        grid=(out_rows // _BR,),
        in_specs=in_specs,
        out_specs=(_row_spec(), _row_spec()),
        out_shape=(_SDS((out_rows, _D), _F32), _SDS((out_rows, _D), _F32)),
    )(deg2, s, *f0_args)


def _update2(deg2, s, t_in, out_rows):
    p = s.shape[0]

    def body(deg2_ref, s_ref, t_ref, o_ref):
        r = _r_of(deg2_ref)
        f2 = r * _sum_parts(s_ref) * (1.0 / 3.0)
        nrm = jnp.sqrt(jnp.sum(f2 * f2, axis=1, keepdims=True))
        o_ref[...] = t_ref[...] + f2 / jnp.maximum(nrm, 1e-12)

    return pl.pallas_call(
        body,
        grid=(out_rows // _BR,),
        in_specs=[_deg2_spec(), _s_spec(p), _row_spec()],
        out_specs=_row_spec(),
        out_shape=_SDS((out_rows, _D), _F32),
    )(deg2, s, t_in)


def _bi_final(deg2, s, out_rows):
    def body(deg2_ref, s_ref, o_ref):
        size = deg2_ref[0, :, 0:1] + deg2_ref[1, :, 0:1] + 1e-8
        o_ref[...] = _sum_parts(s_ref) / size

    return pl.pallas_call(
        body,
        grid=(out_rows // _BR,),
        in_specs=[_deg2_spec(), _s_spec(s.shape[0])],
        out_specs=_row_spec(),
        out_shape=_SDS((out_rows, _D), _F32),
    )(deg2, s)


def _pad_dst(idx, n_dst, e_pad):
    p = e_pad - idx.shape[0]
    fill = jnp.asarray(n_dst + np.arange(p) % 16, jnp.int32)
    return jnp.concatenate([idx.astype(jnp.int32), fill])


def _pad_src(idx, n_src, e_pad):
    p = e_pad - idx.shape[0]
    fill = jnp.asarray(np.arange(p) % n_src, jnp.int32)
    return jnp.concatenate([idx.astype(jnp.int32), fill])


def _branch(a_idx, b_idx, n_a, n_b, feat_a, feat_b_args, zeros, z16, ones16,
            chunk_b):
    e = a_idx.shape[0]
    e_pad = _rup(e, 2 * _KB * _NC * _NS)
    ra, rb = _ra(n_a), _ra(n_b)

    a_dst = _pad_dst(a_idx, n_a, e_pad)
    b_dst = _pad_dst(b_idx, n_b, e_pad)
    a_src = _pad_src(a_idx, n_a, e_pad)
    b_src = _pad_src(b_idx, n_b, e_pad)

    ha, hb = _hist_pair(n_a, n_b, e_pad)(a_dst, b_dst, ones16, z16)
    deg2a = ha.reshape(2, ra, 16)
    deg2b = hb.reshape(2, rb, 16)

    rows_a = _rup(ra, _BR)
    if chunk_b:
        ch = _rup(n_b // 2 + 16, 128)
        rows_b = _rup(2 * ch, _BR)
        pb = 1
    else:
        rows_b = _rup(rb, _BR)
        pb = 2

    ek_a = _edge_prop(n_a, e_pad, False)
    ek_b = _edge_prop(n_b, e_pad, chunk_b)

    if len(feat_b_args) == 2:
        g_b = _scale0_diff(deg2b, feat_b_args[0], feat_b_args[1],
                           _rup(n_b, _BR))
    else:
        g_b = _scale0(deg2b, feat_b_args[0], _rup(n_b, _BR))
    g_a = _scale0(deg2a, feat_a, _rup(n_a, _BR))

    s_a = ek_a(g_b, b_src, a_dst, zeros).reshape(2, ra, _D)
    s_b = ek_b(g_a, a_src, b_dst, zeros)
    s_b = s_b.reshape(pb, s_b.shape[0] // pb, _D)
    t_a, g_a1 = _update1(deg2a, s_a, (feat_a,), rows_a, False)
    t_b, g_b1 = _update1(deg2b, s_b, feat_b_args, rows_b,
                         len(feat_b_args) == 2)

    s_a2 = ek_a(g_b1, b_src, a_dst, zeros).reshape(2, ra, _D)
    s_b2 = ek_b(g_a1, a_src, b_dst, zeros)
    s_b2 = s_b2.reshape(pb, s_b2.shape[0] // pb, _D)
    tot_a = _update2(deg2a, s_a2, t_a, rows_a)
    tot_b = _update2(deg2b, s_b2, t_b, rows_b)
    return tot_a, tot_b


def kernel(users_feat, bundles_feat, items_feat, items_pop,
           ui_u, ui_i, ub_u, ub_b, bi_b, bi_i):
    zeros = jnp.zeros((3200, _D), _F32)
    z16 = zeros.reshape(-1, 16)
    ones16 = jnp.zeros((_KB, 16), _F32).at[:, 0].set(1.0)

    hist_u, hist_b = _branch(ub_u, ub_b, _NU, _NB, users_feat,
                             (bundles_feat,), zeros, z16, ones16, False)
    aff_u, ui_items = _branch(ui_u, ui_i, _NU, _NI, users_feat,
                              (items_feat, items_pop), zeros, z16, ones16,
                              True)

    e_pad = _rup(bi_b.shape[0], 2 * _KB * _NC * _NS)
    b_dst = _pad_dst(bi_b, _NB, e_pad)
    i_src = _pad_src(bi_i, _NI, e_pad)
    hbi = _hist_pair(_NB, None, e_pad)(b_dst, ones16, z16)
    deg2bi = hbi.reshape(2, _ra(_NB), 16)
    s_bi = _edge_prop(_NB, e_pad, False)(ui_items, i_src, b_dst, zeros)
    s_bi = s_bi.reshape(2, _ra(_NB), _D)
    aff_b = _bi_final(deg2bi, s_bi, _rup(_ra(_NB), _BR))

    return (aff_u[:_NU], hist_u[:_NU], aff_b[:_NB], hist_b[:_NB])


# speedup vs baseline: 1.8065x; 1.0144x over previous
import functools

import jax
import jax.numpy as jnp
import numpy as np
from jax import lax
from jax.experimental import pallas as pl
from jax.experimental.pallas import tpu as pltpu
from jax.experimental.pallas import tpu_sc as plsc

_NU, _NB, _NI, _D = 20000, 10000, 50000, 64
_NC, _NS = 2, 16
_KB = 128
_BR = 256
_F32 = jnp.float32
_SDS = jax.ShapeDtypeStruct
_ZROWS = 1600


def _rup(x, m):
    return (x + m - 1) // m * m


def _ra(n):
    return _rup(n + 16, 128)


def _mesh():
    return plsc.VectorSubcoreMesh(core_axis_name="c", subcore_axis_name="s",
                                  num_cores=_NC, num_subcores=_NS)


_SC_PARAMS = pltpu.CompilerParams(use_tc_tiling_on_sc=False)


@functools.lru_cache(maxsize=None)
def _hist_pair(n_a, n_b, e_pad):
    pair = n_b is not None
    ra = _ra(n_a)
    za = ra // 16
    if pair:
        rb = _ra(n_b)
        zb = rb // 16
        out_type = (_SDS((2 * ra, 16), _F32), _SDS((2 * rb, 16), _F32))
    else:
        out_type = _SDS((2 * ra, 16), _F32)
    npt = e_pad // _KB // (_NC * _NS)
    assert npt % 2 == 0 and npt >= 2

    scratch = [
        pltpu.VMEM((_KB,), jnp.int32),
        pltpu.VMEM((_KB,), jnp.int32),
        pltpu.VMEM((_KB, 16), _F32),
        pltpu.VMEM_SHARED((ra, 16), _F32),
        pltpu.SemaphoreType.DMA,
        pltpu.SemaphoreType.DMA,
    ]
    if pair:
        scratch += [
            pltpu.VMEM((_KB,), jnp.int32),
            pltpu.VMEM((_KB,), jnp.int32),
            pltpu.VMEM_SHARED((rb, 16), _F32),
        ]

    @functools.partial(
        pl.kernel,
        out_type=out_type,
        mesh=_mesh(),
        compiler_params=_SC_PARAMS,
        scratch_types=scratch,
    )
    def k(*refs):
        ib_hbm = ob_hbm = ib0 = ib1 = acc_b = None
        if pair:
            (ia_hbm, ib_hbm, ones_hbm, z16_hbm, oa_hbm, ob_hbm,
             ia0, ia1, ones_v, acc_a, sem0, sem1, ib0, ib1, acc_b) = refs
        else:
            (ia_hbm, ones_hbm, z16_hbm, oa_hbm,
             ia0, ia1, ones_v, acc_a, sem0, sem1) = refs
        c = lax.axis_index("c")
        s = lax.axis_index("s")
        w16 = (c * _NS + s) * (_ZROWS * 4)
        pltpu.sync_copy(ones_hbm, ones_v)
        pltpu.sync_copy(z16_hbm.at[pl.ds(w16, za)],
                        acc_a.at[pl.ds(s * za, za)])
        if pair:
            pltpu.sync_copy(z16_hbm.at[pl.ds(w16, zb)],
                            acc_b.at[pl.ds(s * zb, zb)])
        plsc.subcore_barrier()
        base = (c * _NS + s) * npt

        def load(blk, ia, ib):
            pltpu.sync_copy(ia_hbm.at[pl.ds(blk * _KB, _KB)], ia)
            if pair:
                pltpu.sync_copy(ib_hbm.at[pl.ds(blk * _KB, _KB)], ib)

        def fire(ia, ib, sem):
            pltpu.async_copy(ones_v, acc_a.at[ia], sem, add=True)
            if pair:
                pltpu.async_copy(ones_v, acc_b.at[ib], sem, add=True)

        def drain(ia, ib, sem):
            pltpu.make_async_copy(ones_v, acc_a.at[ia], sem).wait()
            if pair:
                pltpu.make_async_copy(ones_v, acc_b.at[ib], sem).wait()

        load(base, ia0, ib0)
        fire(ia0, ib0, sem0)
        load(base + 1, ia1, ib1)
        fire(ia1, ib1, sem1)

        def body(i, carry):
            b0 = base + 2 * i
            drain(ia0, ib0, sem0)
            load(b0 + 2, ia0, ib0)
            fire(ia0, ib0, sem0)
            drain(ia1, ib1, sem1)
            load(b0 + 3, ia1, ib1)
            fire(ia1, ib1, sem1)
            return carry

        lax.fori_loop(0, npt // 2 - 1, body, 0)
        drain(ia0, ib0, sem0)
        drain(ia1, ib1, sem1)
        plsc.subcore_barrier()
        pltpu.sync_copy(acc_a.at[pl.ds(s * za, za)],
                        oa_hbm.at[pl.ds(c * ra + s * za, za)])
        if pair:
            pltpu.sync_copy(acc_b.at[pl.ds(s * zb, zb)],
                            ob_hbm.at[pl.ds(c * rb + s * zb, zb)])

    return k


@functools.lru_cache(maxsize=None)
def _edge_prop(n_dst, e_pad, chunk):
    if chunk:
        ch = _rup(n_dst // 2 + 16, 128)
        acc_rows = ch + 128
        out_rows = 2 * ch
        npt = e_pad // _KB // _NS
        wr = ch // 16
    else:
        acc_rows = _ra(n_dst)
        out_rows = 2 * acc_rows
        npt = e_pad // _KB // (_NC * _NS)
        wr = acc_rows // 16
    zr = acc_rows // 16

    nbuf = 2
    assert npt % nbuf == 0 and npt >= nbuf

    @functools.partial(
        pl.kernel,
        out_type=_SDS((out_rows, _D), _F32),
        mesh=_mesh(),
        compiler_params=_SC_PARAMS,
        scratch_types=(
            [pltpu.VMEM((_KB,), jnp.int32)] * nbuf
            + [pltpu.VMEM((_KB,), jnp.int32)] * nbuf
            + [pltpu.VMEM((_KB, _D), _F32)] * nbuf
            + [pltpu.VMEM_SHARED((acc_rows, _D), _F32)]
            + [pltpu.SemaphoreType.DMA] * nbuf
        ),
    )
    def k(g_hbm, src_hbm, dst_hbm, z_hbm, out_hbm, *scr):
        srcs = scr[0:nbuf]
        dsts = scr[nbuf:2 * nbuf]
        rows = scr[2 * nbuf:3 * nbuf]
        acc = scr[3 * nbuf]
        sems = scr[3 * nbuf + 1:]
        c = lax.axis_index("c")
        s = lax.axis_index("s")
        pltpu.sync_copy(z_hbm.at[pl.ds((c * _NS + s) * _ZROWS, zr)],
                        acc.at[pl.ds(s * zr, zr)])
        plsc.subcore_barrier()
        if chunk:
            base = s * npt
            lo = c * ch
        else:
            base = (c * _NS + s) * npt
        last = base + npt - 1

        def load_start(blk, j):
            pltpu.sync_copy(src_hbm.at[pl.ds(blk * _KB, _KB)], srcs[j])
            pltpu.sync_copy(dst_hbm.at[pl.ds(blk * _KB, _KB)], dsts[j])
            pltpu.async_copy(g_hbm.at[srcs[j]], rows[j], sems[j])

        def gwait(j):
            pltpu.make_async_copy(g_hbm.at[pl.ds(0, _KB)], rows[j],
                                  sems[j]).wait()

        def scat(j):
            dv = dsts[j]
            rf = rows[j]
            if chunk:
                for q in range(_KB // 16):
                    dq = dv[pl.ds(q * 16, 16)]
                    lq = dq - lo
                    ok = (lq >= 0) & (lq < ch)
                    dv[pl.ds(q * 16, 16)] = jnp.where(ok, lq, ch + (dq & 15))
            pltpu.sync_copy(rf, acc.at[dv], add=True)

        for j in range(nbuf):
            load_start(base + j, j)

        def body(i, carry):
            b0 = base + nbuf * i
            for j in range(nbuf):
                gwait(j)
                scat(j)
                load_start(jnp.minimum(b0 + nbuf + j, last), j)
            return carry

        lax.fori_loop(0, npt // nbuf, body, 0)
        for j in range(nbuf):
            gwait(j)
        plsc.subcore_barrier()
        obase = c * (ch if chunk else acc_rows)
        pltpu.sync_copy(acc.at[pl.ds(s * wr, wr)],
                        out_hbm.at[pl.ds(obase + s * wr, wr)])

    return k


def _deg2_spec():
    return pl.BlockSpec((2, _BR, 16), lambda i: (0, i, 0))


def _s_spec(p):
    return pl.BlockSpec((p, _BR, _D), lambda i: (0, i, 0))


def _row_spec():
    return pl.BlockSpec((_BR, _D), lambda i: (i, 0))


def _r_of(deg2_ref):
    d = deg2_ref[0, :, 0:1] + deg2_ref[1, :, 0:1]
    return 1.0 / (jnp.sqrt(d) + 1e-8)


def _scale0(deg2, feat, out_rows):
    def body(deg2_ref, f_ref, o_ref):
        o_ref[...] = f_ref[...] * _r_of(deg2_ref)

    return pl.pallas_call(
        body,
        grid=(out_rows // _BR,),
        in_specs=[_deg2_spec(), _row_spec()],
        out_specs=_row_spec(),
        out_shape=_SDS((out_rows, _D), _F32),
    )(deg2, feat)


def _scale0_diff(deg2, fa, fb, out_rows):
    def body(deg2_ref, fa_ref, fb_ref, o_ref):
        o_ref[...] = (fa_ref[...] - fb_ref[...]) * _r_of(deg2_ref)

    return pl.pallas_call(
        body,
        grid=(out_rows // _BR,),
        in_specs=[_deg2_spec(), _row_spec(), _row_spec()],
        out_specs=_row_spec(),
        out_shape=_SDS((out_rows, _D), _F32),
    )(deg2, fa, fb)


def _sum_parts(s_ref):
    p = s_ref.shape[0]
    acc = s_ref[0]
    for j in range(1, p):
        acc = acc + s_ref[j]
    return acc


def _update1(deg2, s, f0_args, out_rows, diff):
    p = s.shape[0]

    def body(*refs):
        if diff:
            deg2_ref, s_ref, fa_ref, fb_ref, t_ref, g_ref = refs
            f0 = fa_ref[...] - fb_ref[...]
        else:
            deg2_ref, s_ref, fa_ref, t_ref, g_ref = refs
            f0 = fa_ref[...]
        r = _r_of(deg2_ref)
        f1 = r * _sum_parts(s_ref) * 0.5
        nrm = jnp.sqrt(jnp.sum(f1 * f1, axis=1, keepdims=True))
        t_ref[...] = f0 + f1 / jnp.maximum(nrm, 1e-12)
        g_ref[...] = r * f1

    in_specs = [_deg2_spec(), _s_spec(p)] + [_row_spec()] * len(f0_args)
    return pl.pallas_call(
        body,
        grid=(out_rows // _BR,),
        in_specs=in_specs,
        out_specs=(_row_spec(), _row_spec()),
        out_shape=(_SDS((out_rows, _D), _F32), _SDS((out_rows, _D), _F32)),
    )(deg2, s, *f0_args)


def _update2(deg2, s, t_in, out_rows):
    p = s.shape[0]

    def body(deg2_ref, s_ref, t_ref, o_ref):
        r = _r_of(deg2_ref)
        f2 = r * _sum_parts(s_ref) * (1.0 / 3.0)
        nrm = jnp.sqrt(jnp.sum(f2 * f2, axis=1, keepdims=True))
        o_ref[...] = t_ref[...] + f2 / jnp.maximum(nrm, 1e-12)

    return pl.pallas_call(
        body,
        grid=(out_rows // _BR,),
        in_specs=[_deg2_spec(), _s_spec(p), _row_spec()],
        out_specs=_row_spec(),
        out_shape=_SDS((out_rows, _D), _F32),
    )(deg2, s, t_in)


def _bi_final(deg2, s, out_rows):
    def body(deg2_ref, s_ref, o_ref):
        size = deg2_ref[0, :, 0:1] + deg2_ref[1, :, 0:1] + 1e-8
        o_ref[...] = _sum_parts(s_ref) / size

    return pl.pallas_call(
        body,
        grid=(out_rows // _BR,),
        in_specs=[_deg2_spec(), _s_spec(s.shape[0])],
        out_specs=_row_spec(),
        out_shape=_SDS((out_rows, _D), _F32),
    )(deg2, s)


def _pad_dst(idx, n_dst, e_pad):
    p = e_pad - idx.shape[0]
    fill = jnp.asarray(n_dst + np.arange(p) % 16, jnp.int32)
    return jnp.concatenate([idx.astype(jnp.int32), fill])


def _pad_src(idx, n_src, e_pad):
    p = e_pad - idx.shape[0]
    fill = jnp.asarray(np.arange(p) % n_src, jnp.int32)
    return jnp.concatenate([idx.astype(jnp.int32), fill])


def _branch(a_idx, b_idx, n_a, n_b, feat_a, feat_b_args, zeros, z16, ones16,
            chunk_b):
    e = a_idx.shape[0]
    e_pad = _rup(e, 2 * _KB * _NC * _NS)
    ra, rb = _ra(n_a), _ra(n_b)

    a_dst = _pad_dst(a_idx, n_a, e_pad)
    b_dst = _pad_dst(b_idx, n_b, e_pad)
    a_src = _pad_src(a_idx, n_a, e_pad)
    b_src = _pad_src(b_idx, n_b, e_pad)

    ha, hb = _hist_pair(n_a, n_b, e_pad)(a_dst, b_dst, ones16, z16)
    deg2a = ha.reshape(2, ra, 16)
    deg2b = hb.reshape(2, rb, 16)

    rows_a = _rup(ra, _BR)
    if chunk_b:
        ch = _rup(n_b // 2 + 16, 128)
        rows_b = _rup(2 * ch, _BR)
        pb = 1
    else:
        rows_b = _rup(rb, _BR)
        pb = 2

    ek_a = _edge_prop(n_a, e_pad, False)
    ek_b = _edge_prop(n_b, e_pad, chunk_b)

    if len(feat_b_args) == 2:
        g_b = _scale0_diff(deg2b, feat_b_args[0], feat_b_args[1],
                           _rup(n_b, _BR))
    else:
        g_b = _scale0(deg2b, feat_b_args[0], _rup(n_b, _BR))
    g_a = _scale0(deg2a, feat_a, _rup(n_a, _BR))

    s_a = ek_a(g_b, b_src, a_dst, zeros).reshape(2, ra, _D)
    s_b = ek_b(g_a, a_src, b_dst, zeros)
    s_b = s_b.reshape(pb, s_b.shape[0] // pb, _D)
    t_a, g_a1 = _update1(deg2a, s_a, (feat_a,), rows_a, False)
    t_b, g_b1 = _update1(deg2b, s_b, feat_b_args, rows_b,
                         len(feat_b_args) == 2)

    s_a2 = ek_a(g_b1, b_src, a_dst, zeros).reshape(2, ra, _D)
    s_b2 = ek_b(g_a1, a_src, b_dst, zeros)
    s_b2 = s_b2.reshape(pb, s_b2.shape[0] // pb, _D)
    tot_a = _update2(deg2a, s_a2, t_a, rows_a)
    tot_b = _update2(deg2b, s_b2, t_b, rows_b)
    return tot_a, tot_b


def kernel(users_feat, bundles_feat, items_feat, items_pop,
           ui_u, ui_i, ub_u, ub_b, bi_b, bi_i):
    zeros = jnp.zeros((_NC * _NS * _ZROWS, _D), _F32)
    z16 = zeros.reshape(-1, 16)
    ones16 = jnp.zeros((_KB, 16), _F32).at[:, 0].set(1.0)

    hist_u, hist_b = _branch(ub_u, ub_b, _NU, _NB, users_feat,
                             (bundles_feat,), zeros, z16, ones16, False)
    aff_u, ui_items = _branch(ui_u, ui_i, _NU, _NI, users_feat,
                              (items_feat, items_pop), zeros, z16, ones16,
                              True)

    e_pad = _rup(bi_b.shape[0], 2 * _KB * _NC * _NS)
    b_dst = _pad_dst(bi_b, _NB, e_pad)
    i_src = _pad_src(bi_i, _NI, e_pad)
    hbi = _hist_pair(_NB, None, e_pad)(b_dst, ones16, z16)
    deg2bi = hbi.reshape(2, _ra(_NB), 16)
    s_bi = _edge_prop(_NB, e_pad, False)(ui_items, i_src, b_dst, zeros)
    s_bi = s_bi.reshape(2, _ra(_NB), _D)
    aff_b = _bi_final(deg2bi, s_bi, _rup(_ra(_NB), _BR))

    return (aff_u[:_NU], hist_u[:_NU], aff_b[:_NB], hist_b[:_NB])
